# baseline (speedup 1.0000x reference)
import functools

import jax
import jax.numpy as jnp
from jax import lax
from jax.experimental import pallas as pl
from jax.experimental.pallas import tpu as pltpu
from jax.experimental.pallas import tpu_sc as plsc

HID = 768
EPS = 1e-6
NC = 2
NS = 16
NW = NC * NS
TBLK = 128
NSPLIT = 4


@functools.cache
def _build_gather(batch, seq, sseq, h):
    n_tokens = batch * sseq
    rows_per_w = n_tokens // NW
    gchunk = 64 if rows_per_w % 128 == 0 else 32
    nchunks = rows_per_w // gchunk
    assert nchunks % 2 == 0 and sseq % rows_per_w == 0
    sshift = sseq.bit_length() - 1
    mesh = plsc.VectorSubcoreMesh(core_axis_name="c", subcore_axis_name="s")

    @functools.partial(
        pl.kernel,
        mesh=mesh,
        out_type=jax.ShapeDtypeStruct((n_tokens, HID), jnp.float32),
        scratch_types=[
            pltpu.VMEM((rows_per_w,), jnp.int32),
            pltpu.VMEM((gchunk, HID), jnp.float32),
            pltpu.VMEM((gchunk, HID), jnp.float32),
            pltpu.SemaphoreType.DMA,
            pltpu.SemaphoreType.DMA,
            pltpu.SemaphoreType.DMA,
            pltpu.SemaphoreType.DMA,
        ],
    )
    def g(ids_hbm, word_hbm, out_hbm, idx_v, b0, b1, sg0, sg1, so0, so1):
        buf = (b0, b1)
        sg = (sg0, sg1)
        so = (so0, so1)
        wid = lax.axis_index("s") * NC + lax.axis_index("c")
        base = wid * rows_per_w
        b_row = base >> sshift
        soff = base & (sseq - 1)
        flat0 = pl.multiple_of(b_row * seq + h * sseq + soff, gchunk)

        pltpu.sync_copy(ids_hbm.at[pl.ds(flat0, rows_per_w)], idx_v)

        def gather(c, par):
            row0 = pl.multiple_of(c * gchunk, gchunk)
            return pltpu.make_async_copy(
                word_hbm.at[idx_v.at[pl.ds(row0, gchunk)]], buf[par], sg[par])

        def drain(c, par):
            row0 = pl.multiple_of(c * gchunk, gchunk)
            return pltpu.make_async_copy(
                buf[par], out_hbm.at[pl.ds(base + row0, gchunk)], so[par])

        gather(0, 0).start()
        gather(1, 1).start()

        def pair_body(c2, carry):
            c = c2 * 2
            gather(c, 0).wait()
            drain(c, 0).start()

            @pl.when(c + 2 < nchunks)
            def _refill0():
                drain(c, 0).wait()
                gather(c + 2, 0).start()

            gather(c + 1, 1).wait()
            drain(c + 1, 1).start()

            @pl.when(c + 3 < nchunks)
            def _refill1():
                drain(c + 1, 1).wait()
                gather(c + 3, 1).start()

            return carry

        lax.fori_loop(0, nchunks // 2, pair_body, 0)
        drain(nchunks - 2, 0).wait()
        drain(nchunks - 1, 1).wait()

    return g


def _ln_body(x_ref, pos_ref, g_ref, b_ref, o_ref):
    x = x_ref[...] + pos_ref[...][None]
    mean = jnp.mean(x, axis=-1, keepdims=True)
    msq = jnp.mean(x * x, axis=-1, keepdims=True)
    var = msq - mean * mean
    o_ref[...] = (x - mean) * lax.rsqrt(var + EPS) * g_ref[...] + b_ref[...]


def _ln_body_acc(acc_ref, x_ref, pos_ref, g_ref, b_ref, o_ref):
    del acc_ref
    _ln_body(x_ref, pos_ref, g_ref, b_ref, o_ref)


@functools.cache
def _build_ln_chain(batch, seq, nsplit):
    sseq = seq // nsplit
    nj = sseq // TBLK
    out_shape = jax.ShapeDtypeStruct((batch, seq, HID), jnp.float32)
    calls = []
    for h in range(nsplit):
        specs_in = [
            pl.BlockSpec((batch, TBLK, HID), lambda j: (0, j, 0)),
            pl.BlockSpec((TBLK, HID), lambda j, h=h: (j + h * nj, 0)),
            pl.BlockSpec((HID,), lambda j: (0,)),
            pl.BlockSpec((HID,), lambda j: (0,)),
        ]
        out_spec = pl.BlockSpec((batch, TBLK, HID),
                                lambda j, h=h: (0, j + h * nj, 0))
        if h == 0:
            calls.append(pl.pallas_call(
                _ln_body, grid=(nj,), in_specs=specs_in, out_specs=out_spec,
                out_shape=out_shape))
        else:
            calls.append(pl.pallas_call(
                _ln_body_acc, grid=(nj,),
                in_specs=[pl.BlockSpec(memory_space=pl.ANY)] + specs_in,
                out_specs=out_spec, out_shape=out_shape,
                input_output_aliases={0: 0}))
    return calls


def kernel(input_ids, word_embeddings, position_embeddings, gamma, beta):
    b, s = input_ids.shape
    sseq = s // NSPLIT
    ids_flat = input_ids.reshape(-1).astype(jnp.int32)
    pieces = []
    for h in range(NSPLIT):
        pieces.append(_build_gather(b, s, sseq, h)(ids_flat, word_embeddings))
    calls = _build_ln_chain(b, s, NSPLIT)
    out = calls[0](pieces[0].reshape(b, sseq, HID), position_embeddings,
                   gamma, beta)
    for h in range(1, NSPLIT):
        out = calls[h](out, pieces[h].reshape(b, sseq, HID),
                       position_embeddings, gamma, beta)
    return out


# speedup vs baseline: 1.0902x; 1.0902x over previous
import functools

import jax
import jax.numpy as jnp
from jax import lax
from jax.experimental import pallas as pl
from jax.experimental.pallas import tpu as pltpu
from jax.experimental.pallas import tpu_sc as plsc

HID = 768
EPS = 1e-6
NC = 2
NS = 16
NW = NC * NS
TBLK = 128
NSPLIT = 2


@functools.cache
def _build_gather(batch, seq, sseq, h):
    n_tokens = batch * sseq
    rows_per_w = n_tokens // NW
    gchunk = 64 if rows_per_w % 128 == 0 else 32
    nchunks = rows_per_w // gchunk
    assert nchunks % 2 == 0 and sseq % rows_per_w == 0
    sshift = sseq.bit_length() - 1
    mesh = plsc.VectorSubcoreMesh(core_axis_name="c", subcore_axis_name="s")

    @functools.partial(
        pl.kernel,
        mesh=mesh,
        out_type=jax.ShapeDtypeStruct((n_tokens, HID), jnp.float32),
        scratch_types=[
            pltpu.VMEM((rows_per_w,), jnp.int32),
            pltpu.VMEM((gchunk, HID), jnp.float32),
            pltpu.VMEM((gchunk, HID), jnp.float32),
            pltpu.SemaphoreType.DMA,
            pltpu.SemaphoreType.DMA,
            pltpu.SemaphoreType.DMA,
            pltpu.SemaphoreType.DMA,
        ],
    )
    def g(ids_hbm, word_hbm, out_hbm, idx_v, b0, b1, sg0, sg1, so0, so1):
        buf = (b0, b1)
        sg = (sg0, sg1)
        so = (so0, so1)
        wid = lax.axis_index("s") * NC + lax.axis_index("c")
        base = wid * rows_per_w
        b_row = base >> sshift
        soff = base & (sseq - 1)
        flat0 = pl.multiple_of(b_row * seq + h * sseq + soff, gchunk)

        pltpu.sync_copy(ids_hbm.at[pl.ds(flat0, rows_per_w)], idx_v)

        def gather(c, par):
            row0 = pl.multiple_of(c * gchunk, gchunk)
            return pltpu.make_async_copy(
                word_hbm.at[idx_v.at[pl.ds(row0, gchunk)]], buf[par], sg[par])

        def drain(c, par):
            row0 = pl.multiple_of(c * gchunk, gchunk)
            return pltpu.make_async_copy(
                buf[par], out_hbm.at[pl.ds(base + row0, gchunk)], so[par])

        gather(0, 0).start()
        gather(1, 1).start()

        def pair_body(c2, carry):
            c = c2 * 2
            gather(c, 0).wait()
            drain(c, 0).start()

            @pl.when(c + 2 < nchunks)
            def _refill0():
                drain(c, 0).wait()
                gather(c + 2, 0).start()

            gather(c + 1, 1).wait()
            drain(c + 1, 1).start()

            @pl.when(c + 3 < nchunks)
            def _refill1():
                drain(c + 1, 1).wait()
                gather(c + 3, 1).start()

            return carry

        lax.fori_loop(0, nchunks // 2, pair_body, 0)
        drain(nchunks - 2, 0).wait()
        drain(nchunks - 1, 1).wait()

    return g


def _ln_body(x_ref, pos_ref, g_ref, b_ref, o_ref):
    x = x_ref[...] + pos_ref[...][None]
    mean = jnp.mean(x, axis=-1, keepdims=True)
    msq = jnp.mean(x * x, axis=-1, keepdims=True)
    var = msq - mean * mean
    o_ref[...] = (x - mean) * lax.rsqrt(var + EPS) * g_ref[...] + b_ref[...]


def _ln_body_acc(acc_ref, x_ref, pos_ref, g_ref, b_ref, o_ref):
    del acc_ref
    _ln_body(x_ref, pos_ref, g_ref, b_ref, o_ref)


@functools.cache
def _build_ln_chain(batch, seq, nsplit):
    sseq = seq // nsplit
    nj = sseq // TBLK
    out_shape = jax.ShapeDtypeStruct((batch, seq, HID), jnp.float32)
    calls = []
    for h in range(nsplit):
        specs_in = [
            pl.BlockSpec((batch, TBLK, HID), lambda j: (0, j, 0)),
            pl.BlockSpec((TBLK, HID), lambda j, h=h: (j + h * nj, 0)),
            pl.BlockSpec((HID,), lambda j: (0,)),
            pl.BlockSpec((HID,), lambda j: (0,)),
        ]
        out_spec = pl.BlockSpec((batch, TBLK, HID),
                                lambda j, h=h: (0, j + h * nj, 0))
        if h == 0:
            calls.append(pl.pallas_call(
                _ln_body, grid=(nj,), in_specs=specs_in, out_specs=out_spec,
                out_shape=out_shape))
        else:
            calls.append(pl.pallas_call(
                _ln_body_acc, grid=(nj,),
                in_specs=[pl.BlockSpec(memory_space=pl.ANY)] + specs_in,
                out_specs=out_spec, out_shape=out_shape,
                input_output_aliases={0: 0}))
    return calls


def kernel(input_ids, word_embeddings, position_embeddings, gamma, beta):
    b, s = input_ids.shape
    sseq = s // NSPLIT
    ids_flat = input_ids.reshape(-1).astype(jnp.int32)
    pieces = []
    for h in range(NSPLIT):
        pieces.append(_build_gather(b, s, sseq, h)(ids_flat, word_embeddings))
    calls = _build_ln_chain(b, s, NSPLIT)
    out = calls[0](pieces[0].reshape(b, sseq, HID), position_embeddings,
                   gamma, beta)
    for h in range(1, NSPLIT):
        out = calls[h](out, pieces[h].reshape(b, sseq, HID),
                       position_embeddings, gamma, beta)
    return out


# speedup vs baseline: 1.1162x; 1.0238x over previous
import functools

import jax
import jax.numpy as jnp
from jax import lax
from jax.experimental import pallas as pl
from jax.experimental.pallas import tpu as pltpu
from jax.experimental.pallas import tpu_sc as plsc

HID = 768
EPS = 1e-6
NC = 2
NS = 16
NW = NC * NS
TBLK = 256
NSPLIT = 1


@functools.cache
def _build_gather(batch, seq, sseq, h):
    n_tokens = batch * sseq
    rows_per_w = n_tokens // NW
    gchunk = 64 if rows_per_w % 128 == 0 else 32
    nchunks = rows_per_w // gchunk
    assert nchunks % 2 == 0 and sseq % rows_per_w == 0
    sshift = sseq.bit_length() - 1
    mesh = plsc.VectorSubcoreMesh(core_axis_name="c", subcore_axis_name="s")

    @functools.partial(
        pl.kernel,
        mesh=mesh,
        out_type=jax.ShapeDtypeStruct((n_tokens, HID), jnp.float32),
        scratch_types=[
            pltpu.VMEM((rows_per_w,), jnp.int32),
            pltpu.VMEM((gchunk, HID), jnp.float32),
            pltpu.VMEM((gchunk, HID), jnp.float32),
            pltpu.SemaphoreType.DMA,
            pltpu.SemaphoreType.DMA,
            pltpu.SemaphoreType.DMA,
            pltpu.SemaphoreType.DMA,
        ],
    )
    def g(ids_hbm, word_hbm, out_hbm, idx_v, b0, b1, sg0, sg1, so0, so1):
        buf = (b0, b1)
        sg = (sg0, sg1)
        so = (so0, so1)
        wid = lax.axis_index("s") * NC + lax.axis_index("c")
        base = wid * rows_per_w
        b_row = base >> sshift
        soff = base & (sseq - 1)
        flat0 = pl.multiple_of(b_row * seq + h * sseq + soff, gchunk)

        pltpu.sync_copy(ids_hbm.at[pl.ds(flat0, rows_per_w)], idx_v)

        def gather(c, par):
            row0 = pl.multiple_of(c * gchunk, gchunk)
            return pltpu.make_async_copy(
                word_hbm.at[idx_v.at[pl.ds(row0, gchunk)]], buf[par], sg[par])

        def drain(c, par):
            row0 = pl.multiple_of(c * gchunk, gchunk)
            return pltpu.make_async_copy(
                buf[par], out_hbm.at[pl.ds(base + row0, gchunk)], so[par])

        gather(0, 0).start()
        gather(1, 1).start()

        def pair_body(c2, carry):
            c = c2 * 2
            gather(c, 0).wait()
            drain(c, 0).start()

            @pl.when(c + 2 < nchunks)
            def _refill0():
                drain(c, 0).wait()
                gather(c + 2, 0).start()

            gather(c + 1, 1).wait()
            drain(c + 1, 1).start()

            @pl.when(c + 3 < nchunks)
            def _refill1():
                drain(c + 1, 1).wait()
                gather(c + 3, 1).start()

            return carry

        lax.fori_loop(0, nchunks // 2, pair_body, 0)
        drain(nchunks - 2, 0).wait()
        drain(nchunks - 1, 1).wait()

    return g


def _ln_body(x_ref, pos_ref, g_ref, b_ref, o_ref):
    x = x_ref[...] + pos_ref[...][None]
    mean = jnp.mean(x, axis=-1, keepdims=True)
    msq = jnp.mean(x * x, axis=-1, keepdims=True)
    var = msq - mean * mean
    o_ref[...] = (x - mean) * lax.rsqrt(var + EPS) * g_ref[...] + b_ref[...]


def _ln_body_acc(acc_ref, x_ref, pos_ref, g_ref, b_ref, o_ref):
    del acc_ref
    _ln_body(x_ref, pos_ref, g_ref, b_ref, o_ref)


@functools.cache
def _build_ln_chain(batch, seq, nsplit):
    sseq = seq // nsplit
    nj = sseq // TBLK
    out_shape = jax.ShapeDtypeStruct((batch, seq, HID), jnp.float32)
    calls = []
    for h in range(nsplit):
        specs_in = [
            pl.BlockSpec((batch, TBLK, HID), lambda j: (0, j, 0)),
            pl.BlockSpec((TBLK, HID), lambda j, h=h: (j + h * nj, 0)),
            pl.BlockSpec((HID,), lambda j: (0,)),
            pl.BlockSpec((HID,), lambda j: (0,)),
        ]
        out_spec = pl.BlockSpec((batch, TBLK, HID),
                                lambda j, h=h: (0, j + h * nj, 0))
        if h == 0:
            calls.append(pl.pallas_call(
                _ln_body, grid=(nj,), in_specs=specs_in, out_specs=out_spec,
                out_shape=out_shape))
        else:
            calls.append(pl.pallas_call(
                _ln_body_acc, grid=(nj,),
                in_specs=[pl.BlockSpec(memory_space=pl.ANY)] + specs_in,
                out_specs=out_spec, out_shape=out_shape,
                input_output_aliases={0: 0}))
    return calls


def kernel(input_ids, word_embeddings, position_embeddings, gamma, beta):
    b, s = input_ids.shape
    sseq = s // NSPLIT
    ids_flat = input_ids.reshape(-1).astype(jnp.int32)
    pieces = []
    for h in range(NSPLIT):
        pieces.append(_build_gather(b, s, sseq, h)(ids_flat, word_embeddings))
    calls = _build_ln_chain(b, s, NSPLIT)
    out = calls[0](pieces[0].reshape(b, sseq, HID), position_embeddings,
                   gamma, beta)
    for h in range(1, NSPLIT):
        out = calls[h](out, pieces[h].reshape(b, sseq, HID),
                       position_embeddings, gamma, beta)
    return out


# speedup vs baseline: 1.1335x; 1.0156x over previous
import functools

import jax
import jax.numpy as jnp
from jax import lax
from jax.experimental import pallas as pl
from jax.experimental.pallas import tpu as pltpu
from jax.experimental.pallas import tpu_sc as plsc

HID = 768
EPS = 1e-6
NC = 2
NS = 16
NW = NC * NS
TBLK = 256
NSPLIT = 1


@functools.cache
def _build_gather(batch, seq, sseq, h):
    n_tokens = batch * sseq
    rows_per_w = n_tokens // NW
    gchunk = 32
    nbuf = 4
    nchunks = rows_per_w // gchunk
    assert nchunks % nbuf == 0 and sseq % rows_per_w == 0
    sshift = sseq.bit_length() - 1
    mesh = plsc.VectorSubcoreMesh(core_axis_name="c", subcore_axis_name="s")

    @functools.partial(
        pl.kernel,
        mesh=mesh,
        out_type=jax.ShapeDtypeStruct((n_tokens, HID), jnp.float32),
        scratch_types=(
            [pltpu.VMEM((rows_per_w,), jnp.int32)]
            + [pltpu.VMEM((gchunk, HID), jnp.float32)] * nbuf
            + [pltpu.SemaphoreType.DMA] * (2 * nbuf)
        ),
    )
    def g(ids_hbm, word_hbm, out_hbm, idx_v, *rest):
        buf = rest[:nbuf]
        sg = rest[nbuf:2 * nbuf]
        so = rest[2 * nbuf:]
        wid = lax.axis_index("s") * NC + lax.axis_index("c")
        base = wid * rows_per_w
        b_row = base >> sshift
        soff = base & (sseq - 1)
        flat0 = pl.multiple_of(b_row * seq + h * sseq + soff, gchunk)

        pltpu.sync_copy(ids_hbm.at[pl.ds(flat0, rows_per_w)], idx_v)

        def gather(c, par):
            row0 = pl.multiple_of(c * gchunk, gchunk)
            return pltpu.make_async_copy(
                word_hbm.at[idx_v.at[pl.ds(row0, gchunk)]], buf[par], sg[par])

        def drain(c, par):
            row0 = pl.multiple_of(c * gchunk, gchunk)
            return pltpu.make_async_copy(
                buf[par], out_hbm.at[pl.ds(base + row0, gchunk)], so[par])

        for c in range(nbuf):
            gather(c, c).start()
        for c in range(nchunks):
            par = c % nbuf
            if c >= 1 and c - 1 + nbuf < nchunks:
                drain(c - 1, (c - 1) % nbuf).wait()
                gather(c - 1 + nbuf, (c - 1) % nbuf).start()
            gather(c, par).wait()
            drain(c, par).start()
        for c in range(nchunks - nbuf, nchunks):
            drain(c, c % nbuf).wait()

    return g


def _ln_body(x_ref, pos_ref, g_ref, b_ref, o_ref):
    x = x_ref[...] + pos_ref[...][None]
    mean = jnp.mean(x, axis=-1, keepdims=True)
    msq = jnp.mean(x * x, axis=-1, keepdims=True)
    var = msq - mean * mean
    o_ref[...] = (x - mean) * lax.rsqrt(var + EPS) * g_ref[...] + b_ref[...]


def _ln_body_acc(acc_ref, x_ref, pos_ref, g_ref, b_ref, o_ref):
    del acc_ref
    _ln_body(x_ref, pos_ref, g_ref, b_ref, o_ref)


@functools.cache
def _build_ln_chain(batch, seq, nsplit):
    sseq = seq // nsplit
    nj = sseq // TBLK
    out_shape = jax.ShapeDtypeStruct((batch, seq, HID), jnp.float32)
    calls = []
    for h in range(nsplit):
        specs_in = [
            pl.BlockSpec((batch, TBLK, HID), lambda j: (0, j, 0)),
            pl.BlockSpec((TBLK, HID), lambda j, h=h: (j + h * nj, 0)),
            pl.BlockSpec((HID,), lambda j: (0,)),
            pl.BlockSpec((HID,), lambda j: (0,)),
        ]
        out_spec = pl.BlockSpec((batch, TBLK, HID),
                                lambda j, h=h: (0, j + h * nj, 0))
        if h == 0:
            calls.append(pl.pallas_call(
                _ln_body, grid=(nj,), in_specs=specs_in, out_specs=out_spec,
                out_shape=out_shape))
        else:
            calls.append(pl.pallas_call(
                _ln_body_acc, grid=(nj,),
                in_specs=[pl.BlockSpec(memory_space=pl.ANY)] + specs_in,
                out_specs=out_spec, out_shape=out_shape,
                input_output_aliases={0: 0}))
    return calls


def kernel(input_ids, word_embeddings, position_embeddings, gamma, beta):
    b, s = input_ids.shape
    sseq = s // NSPLIT
    ids_flat = input_ids.reshape(-1).astype(jnp.int32)
    pieces = []
    for h in range(NSPLIT):
        pieces.append(_build_gather(b, s, sseq, h)(ids_flat, word_embeddings))
    calls = _build_ln_chain(b, s, NSPLIT)
    out = calls[0](pieces[0].reshape(b, sseq, HID), position_embeddings,
                   gamma, beta)
    for h in range(1, NSPLIT):
        out = calls[h](out, pieces[h].reshape(b, sseq, HID),
                       position_embeddings, gamma, beta)
    return out


# speedup vs baseline: 1.1722x; 1.0341x over previous
import functools

import jax
import jax.numpy as jnp
from jax import lax
from jax.experimental import pallas as pl
from jax.experimental.pallas import tpu as pltpu
from jax.experimental.pallas import tpu_sc as plsc

HID = 768
EPS = 1e-6
NC = 2
NS = 16
NW = NC * NS
TBLK = 512
NSPLIT = 1


@functools.cache
def _build_gather(batch, seq, sseq, h):
    n_tokens = batch * sseq
    rows_per_w = n_tokens // NW
    gchunk = 32
    nbuf = 4
    nchunks = rows_per_w // gchunk
    assert nchunks % nbuf == 0 and sseq % rows_per_w == 0
    sshift = sseq.bit_length() - 1
    mesh = plsc.VectorSubcoreMesh(core_axis_name="c", subcore_axis_name="s")

    @functools.partial(
        pl.kernel,
        mesh=mesh,
        out_type=jax.ShapeDtypeStruct((n_tokens, HID), jnp.float32),
        scratch_types=(
            [pltpu.VMEM((rows_per_w,), jnp.int32)]
            + [pltpu.VMEM((gchunk, HID), jnp.float32)] * nbuf
            + [pltpu.SemaphoreType.DMA] * (2 * nbuf)
        ),
    )
    def g(ids_hbm, word_hbm, out_hbm, idx_v, *rest):
        buf = rest[:nbuf]
        sg = rest[nbuf:2 * nbuf]
        so = rest[2 * nbuf:]
        wid = lax.axis_index("s") * NC + lax.axis_index("c")
        base = wid * rows_per_w
        b_row = base >> sshift
        soff = base & (sseq - 1)
        flat0 = pl.multiple_of(b_row * seq + h * sseq + soff, gchunk)

        pltpu.sync_copy(ids_hbm.at[pl.ds(flat0, rows_per_w)], idx_v)

        def gather(c, par):
            row0 = pl.multiple_of(c * gchunk, gchunk)
            return pltpu.make_async_copy(
                word_hbm.at[idx_v.at[pl.ds(row0, gchunk)]], buf[par], sg[par])

        def drain(c, par):
            row0 = pl.multiple_of(c * gchunk, gchunk)
            return pltpu.make_async_copy(
                buf[par], out_hbm.at[pl.ds(base + row0, gchunk)], so[par])

        for c in range(nbuf):
            gather(c, c).start()
        for c in range(nchunks):
            par = c % nbuf
            if c >= 1 and c - 1 + nbuf < nchunks:
                drain(c - 1, (c - 1) % nbuf).wait()
                gather(c - 1 + nbuf, (c - 1) % nbuf).start()
            gather(c, par).wait()
            drain(c, par).start()
        for c in range(nchunks - nbuf, nchunks):
            drain(c, c % nbuf).wait()

    return g


def _ln_body(x_ref, pos_ref, g_ref, b_ref, o_ref):
    x = x_ref[...] + pos_ref[...][None]
    mean = jnp.mean(x, axis=-1, keepdims=True)
    msq = jnp.mean(x * x, axis=-1, keepdims=True)
    var = msq - mean * mean
    o_ref[...] = (x - mean) * lax.rsqrt(var + EPS) * g_ref[...] + b_ref[...]


def _ln_body_acc(acc_ref, x_ref, pos_ref, g_ref, b_ref, o_ref):
    del acc_ref
    _ln_body(x_ref, pos_ref, g_ref, b_ref, o_ref)


@functools.cache
def _build_ln_chain(batch, seq, nsplit):
    sseq = seq // nsplit
    nj = sseq // TBLK
    out_shape = jax.ShapeDtypeStruct((batch, seq, HID), jnp.float32)
    calls = []
    for h in range(nsplit):
        specs_in = [
            pl.BlockSpec((batch, TBLK, HID), lambda j: (0, j, 0)),
            pl.BlockSpec((TBLK, HID), lambda j, h=h: (j + h * nj, 0)),
            pl.BlockSpec((HID,), lambda j: (0,)),
            pl.BlockSpec((HID,), lambda j: (0,)),
        ]
        out_spec = pl.BlockSpec((batch, TBLK, HID),
                                lambda j, h=h: (0, j + h * nj, 0))
        if h == 0:
            calls.append(pl.pallas_call(
                _ln_body, grid=(nj,), in_specs=specs_in, out_specs=out_spec,
                out_shape=out_shape))
        else:
            calls.append(pl.pallas_call(
                _ln_body_acc, grid=(nj,),
                in_specs=[pl.BlockSpec(memory_space=pl.ANY)] + specs_in,
                out_specs=out_spec, out_shape=out_shape,
                input_output_aliases={0: 0}))
    return calls


def kernel(input_ids, word_embeddings, position_embeddings, gamma, beta):
    b, s = input_ids.shape
    sseq = s // NSPLIT
    ids_flat = input_ids.reshape(-1).astype(jnp.int32)
    pieces = []
    for h in range(NSPLIT):
        pieces.append(_build_gather(b, s, sseq, h)(ids_flat, word_embeddings))
    calls = _build_ln_chain(b, s, NSPLIT)
    out = calls[0](pieces[0].reshape(b, sseq, HID), position_embeddings,
                   gamma, beta)
    for h in range(1, NSPLIT):
        out = calls[h](out, pieces[h].reshape(b, sseq, HID),
                       position_embeddings, gamma, beta)
    return out
